# Initial kernel scaffold; baseline (speedup 1.0000x reference)
import jax
import jax.numpy as jnp
from jax.experimental import pallas as pl


def kernel(idx, table):
    raise NotImplementedError("write your pallas kernel here")


# speedup vs baseline: 4.7419x; 4.7419x over previous
import functools

import jax
import jax.numpy as jnp
from jax import lax
from jax.experimental import pallas as pl
from jax.experimental.pallas import tpu as pltpu
from jax.experimental.pallas import tpu_sc as plsc

_BATCH = 16384
_SEQ = 200
_D = 64
_B = _BATCH * _SEQ

_NC = 2
_NS = 16
_NW = _NC * _NS

_B_PER_W = _B // _NW
_CHUNK = 512
_NCH = _B_PER_W // _CHUNK

_mesh = plsc.VectorSubcoreMesh(
    core_axis_name="c", subcore_axis_name="s", num_cores=_NC, num_subcores=_NS
)


@functools.partial(
    pl.kernel,
    out_type=jax.ShapeDtypeStruct((_B, _D), jnp.float32),
    mesh=_mesh,
    scratch_types=[
        pltpu.VMEM((_CHUNK,), jnp.int32),
        pltpu.VMEM((_CHUNK, _D), jnp.float32),
        pltpu.SemaphoreType.DMA,
    ],
    compiler_params=pltpu.CompilerParams(use_tc_tiling_on_sc=False),
)
def _embed_sc(idx_hbm, table_hbm, out_hbm, idx_v, rows_v, sem):
    wid = lax.axis_index("s") * _NC + lax.axis_index("c")
    base = wid * _B_PER_W

    def body(g, carry):
        off = base + g * _CHUNK
        pltpu.sync_copy(idx_hbm.at[pl.ds(off, _CHUNK)], idx_v)
        pltpu.async_copy(table_hbm.at[idx_v], rows_v, sem).wait()
        pltpu.sync_copy(rows_v, out_hbm.at[pl.ds(off, _CHUNK)])
        return carry

    lax.fori_loop(0, _NCH, body, 0)


def kernel(idx, table):
    out = _embed_sc(idx.reshape(_B), table)
    return out.reshape(_BATCH, _SEQ, _D)


# speedup vs baseline: 5.1804x; 1.0925x over previous
import functools

import jax
import jax.numpy as jnp
from jax import lax
from jax.experimental import pallas as pl
from jax.experimental.pallas import tpu as pltpu
from jax.experimental.pallas import tpu_sc as plsc

_BATCH = 16384
_SEQ = 200
_D = 64
_B = _BATCH * _SEQ

_NC = 2
_NS = 16
_NW = _NC * _NS

_B_PER_W = _B // _NW
_CHUNK = 400
_NCH = _B_PER_W // _CHUNK
_NBUF = 4

_mesh = plsc.VectorSubcoreMesh(
    core_axis_name="c", subcore_axis_name="s", num_cores=_NC, num_subcores=_NS
)


@functools.partial(
    pl.kernel,
    out_type=jax.ShapeDtypeStruct((_B, _D), jnp.float32),
    mesh=_mesh,
    scratch_types=[
        [pltpu.VMEM((_CHUNK,), jnp.int32) for _ in range(_NBUF)],
        [pltpu.VMEM((_CHUNK, _D), jnp.float32) for _ in range(_NBUF)],
        [pltpu.SemaphoreType.DMA for _ in range(_NBUF)],
        [pltpu.SemaphoreType.DMA for _ in range(_NBUF)],
    ],
    compiler_params=pltpu.CompilerParams(use_tc_tiling_on_sc=False),
)
def _embed_sc(idx_hbm, table_hbm, out_hbm, idx_v, rows_v, sem_g, sem_o):
    wid = lax.axis_index("s") * _NC + lax.axis_index("c")
    base = wid * _B_PER_W

    def start_gather(g, slot):
        off = base + g * _CHUNK
        pltpu.sync_copy(idx_hbm.at[pl.ds(off, _CHUNK)], idx_v[slot])
        pltpu.async_copy(table_hbm.at[idx_v[slot]], rows_v[slot], sem_g[slot])

    def start_out(g, slot):
        off = base + g * _CHUNK
        pltpu.async_copy(rows_v[slot], out_hbm.at[pl.ds(off, _CHUNK)], sem_o[slot])

    def wait_gather(slot):
        pltpu.make_async_copy(
            table_hbm.at[idx_v[slot]], rows_v[slot], sem_g[slot]
        ).wait()

    def wait_out(slot):
        pltpu.make_async_copy(
            rows_v[slot], out_hbm.at[pl.ds(0, _CHUNK)], sem_o[slot]
        ).wait()

    for j in range(_NBUF):
        start_gather(j, j)
    for g in (0, 1):
        wait_gather(g)
        start_out(g, g)

    def body(gg, carry):
        for j in range(_NBUF):
            g = 2 + gg * _NBUF + j
            b = (2 + j) % _NBUF
            b2 = j
            wait_gather(b)
            start_out(g, b)
            wait_out(b2)
            start_gather(g + 2, b2)
        return carry

    lax.fori_loop(0, (_NCH - 4) // _NBUF, body, 0)

    for g in (_NCH - 2, _NCH - 1):
        b = g % _NBUF
        wait_gather(b)
        start_out(g, b)
    for j in range(_NBUF):
        wait_out(j)


def kernel(idx, table):
    out = _embed_sc(idx.reshape(_B), table)
    return out.reshape(_BATCH, _SEQ, _D)
